# Initial kernel scaffold; baseline (speedup 1.0000x reference)
import jax
import jax.numpy as jnp
from jax.experimental import pallas as pl


def kernel(node_input, labels):
    raise NotImplementedError("write your pallas kernel here")


# speedup vs baseline: 289.7496x; 289.7496x over previous
import functools

import jax
import jax.numpy as jnp
from jax import lax
from jax.experimental import pallas as pl
from jax.experimental.pallas import tpu as pltpu
from jax.experimental.pallas import tpu_sc as plsc

N = 320000
DIM = 128
NSEG = 10000
NSEG_PAD = 10240
NC = 2
NS = 16
NW = NC * NS
ROWS_PER_TILE = N // NW
CHUNK = 80
NCHUNK = ROWS_PER_TILE // CHUNK
SEG_PER_TILE = NSEG_PAD // NS
CNT_PER_TILE = NSEG_PAD // NS
ZROWS = 128


def _phase_a(node_input, labels):
    mesh = plsc.VectorSubcoreMesh(core_axis_name="c", subcore_axis_name="s")

    @functools.partial(
        pl.kernel,
        out_type=[
            jax.ShapeDtypeStruct((NC, NSEG_PAD, DIM), jnp.float32),
            jax.ShapeDtypeStruct((NC, NSEG_PAD), jnp.float32),
        ],
        mesh=mesh,
        scratch_types=[
            pltpu.VMEM_SHARED((NSEG_PAD, DIM), jnp.float32),
            pltpu.VMEM_SHARED((NSEG_PAD,), jnp.float32),
            pltpu.VMEM((CHUNK, DIM), jnp.float32),
            pltpu.VMEM((CHUNK,), jnp.int32),
            pltpu.VMEM((CHUNK,), jnp.float32),
            pltpu.VMEM((ZROWS, DIM), jnp.float32),
            pltpu.VMEM((CNT_PER_TILE,), jnp.float32),
        ],
    )
    def k(node_h, lab_h, sums_h, cnt_h, acc, cacc, rbuf, lbuf, ones, zrow, zcnt):
        cid = lax.axis_index("c")
        sid = lax.axis_index("s")
        row0 = cid * (N // NC) + sid * ROWS_PER_TILE

        zeros16 = jnp.zeros((16,), jnp.float32)
        ones16 = jnp.ones((16,), jnp.float32)

        def fill_zrow(i, carry):
            for j in range(DIM // 16):
                zrow[i, pl.ds(j * 16, 16)] = zeros16
            return carry

        lax.fori_loop(0, ZROWS, fill_zrow, 0)

        def fill_zcnt(i, carry):
            zcnt[pl.ds(i * 16, 16)] = zeros16
            return carry

        lax.fori_loop(0, CNT_PER_TILE // 16, fill_zcnt, 0)

        for j in range(CHUNK // 16):
            ones[pl.ds(j * 16, 16)] = ones16

        seg0 = sid * SEG_PER_TILE
        nfull = SEG_PER_TILE // ZROWS
        rem = SEG_PER_TILE - nfull * ZROWS
        for j in range(nfull):
            pltpu.sync_copy(zrow, acc.at[pl.ds(seg0 + j * ZROWS, ZROWS)])
        if rem:
            pltpu.sync_copy(zrow.at[pl.ds(0, rem)],
                            acc.at[pl.ds(seg0 + nfull * ZROWS, rem)])
        pltpu.sync_copy(zcnt, cacc.at[pl.ds(sid * CNT_PER_TILE, CNT_PER_TILE)])

        plsc.subcore_barrier()

        def chunk_body(kk, carry):
            base = row0 + kk * CHUNK
            pltpu.sync_copy(node_h.at[pl.ds(base, CHUNK)], rbuf)
            pltpu.sync_copy(lab_h.at[pl.ds(base, CHUNK)], lbuf)
            pltpu.sync_copy(rbuf, acc.at[lbuf], add=True)
            pltpu.sync_copy(ones, cacc.at[lbuf], add=True)
            return carry

        lax.fori_loop(0, NCHUNK, chunk_body, 0)

        plsc.subcore_barrier()

        pltpu.sync_copy(acc.at[pl.ds(seg0, SEG_PER_TILE)],
                        sums_h.at[cid, pl.ds(seg0, SEG_PER_TILE)])
        pltpu.sync_copy(cacc.at[pl.ds(sid * CNT_PER_TILE, CNT_PER_TILE)],
                        cnt_h.at[cid, pl.ds(sid * CNT_PER_TILE, CNT_PER_TILE)])

    return k(node_input, labels)


def _combine_body(sums_ref, cnt_ref, out_ref):
    s = sums_ref[0, :NSEG] + sums_ref[1, :NSEG]
    c = jnp.maximum(cnt_ref[0, :NSEG] + cnt_ref[1, :NSEG], 1.0)
    out_ref[...] = s / c[:, None]


def kernel(node_input, labels):
    labels = labels.astype(jnp.int32)
    sums, cnts = _phase_a(node_input, labels)
    out = pl.pallas_call(
        _combine_body,
        out_shape=jax.ShapeDtypeStruct((NSEG, DIM), jnp.float32),
    )(sums, cnts)
    return out


# speedup vs baseline: 598.2356x; 2.0647x over previous
import functools

import jax
import jax.numpy as jnp
from jax import lax
from jax.experimental import pallas as pl
from jax.experimental.pallas import tpu as pltpu
from jax.experimental.pallas import tpu_sc as plsc

N = 320000
DIM = 128
NSEG = 10000
NSEG_PAD = 10240
NC = 2
NS = 16
NW = NC * NS
ROWS_PER_TILE = N // NW
CHUNK = 88
NCHUNK = ROWS_PER_TILE // CHUNK
TAIL = ROWS_PER_TILE - NCHUNK * CHUNK
SEG_PER_TILE = NSEG_PAD // NS
CNT_PER_TILE = NSEG_PAD // NS
ZROWS = 128
NBUF = 4


def _phase_a(node_input, labels):
    mesh = plsc.VectorSubcoreMesh(core_axis_name="c", subcore_axis_name="s")

    @functools.partial(
        pl.kernel,
        out_type=[
            jax.ShapeDtypeStruct((NC, NSEG_PAD, DIM), jnp.float32),
            jax.ShapeDtypeStruct((NC, NSEG_PAD), jnp.float32),
        ],
        mesh=mesh,
        scratch_types=[
            pltpu.VMEM_SHARED((NSEG_PAD, DIM), jnp.float32),
            pltpu.VMEM_SHARED((NSEG_PAD,), jnp.float32),
            pltpu.VMEM((NBUF * CHUNK, DIM), jnp.float32),
            pltpu.VMEM((NBUF, CHUNK), jnp.int32),
            pltpu.VMEM((CHUNK,), jnp.float32),
            pltpu.VMEM((TAIL,), jnp.int32),
            pltpu.VMEM((CNT_PER_TILE,), jnp.float32),
            pltpu.SemaphoreType.DMA((NBUF,)),
            pltpu.SemaphoreType.DMA((NBUF,)),
        ],
    )
    def k(node_h, lab_h, sums_h, cnt_h, acc, cacc, rbuf, lbuf, ones, tlab,
          zcnt, gsem, ssem):
        cid = lax.axis_index("c")
        sid = lax.axis_index("s")
        row0 = cid * (N // NC) + sid * ROWS_PER_TILE

        zeros16 = jnp.zeros((16,), jnp.float32)
        ones16 = jnp.ones((16,), jnp.float32)

        def fill_zrow(i, carry):
            for j in range(DIM // 16):
                rbuf[i, pl.ds(j * 16, 16)] = zeros16
            return carry

        lax.fori_loop(0, NBUF * CHUNK, fill_zrow, 0)

        def fill_zcnt(i, carry):
            zcnt[pl.ds(i * 16, 16)] = zeros16
            return carry

        lax.fori_loop(0, CNT_PER_TILE // 16, fill_zcnt, 0)

        for j in range(CHUNK // 16):
            ones[pl.ds(j * 16, 16)] = ones16

        seg0 = sid * SEG_PER_TILE
        zr = NBUF * CHUNK
        done = 0
        while done < SEG_PER_TILE:
            step = min(zr, SEG_PER_TILE - done)
            pltpu.sync_copy(rbuf.at[pl.ds(0, step)],
                            acc.at[pl.ds(seg0 + done, step)])
            done += step
        pltpu.sync_copy(zcnt, cacc.at[pl.ds(sid * CNT_PER_TILE, CNT_PER_TILE)])

        plsc.subcore_barrier()

        tbase = row0 + NCHUNK * CHUNK
        pltpu.sync_copy(node_h.at[pl.ds(tbase, TAIL)], rbuf.at[pl.ds(0, TAIL)])
        pltpu.sync_copy(lab_h.at[pl.ds(tbase, TAIL)], tlab)
        pltpu.sync_copy(rbuf.at[pl.ds(0, TAIL)], acc.at[tlab], add=True)
        pltpu.sync_copy(ones.at[pl.ds(0, TAIL)], cacc.at[tlab], add=True)

        def rslot(b):
            return rbuf.at[pl.ds(b * CHUNK, CHUNK)]

        def start_gather(kk, b):
            base = row0 + kk * CHUNK
            pltpu.async_copy(node_h.at[pl.ds(base, CHUNK)], rslot(b),
                             gsem.at[b])
            pltpu.async_copy(lab_h.at[pl.ds(base, CHUNK)], lbuf.at[b],
                             gsem.at[b])

        def wait_gather(b):
            pltpu.make_async_copy(node_h.at[pl.ds(0, CHUNK)], rslot(b),
                                  gsem.at[b]).wait()
            pltpu.make_async_copy(lab_h.at[pl.ds(0, CHUNK)], lbuf.at[b],
                                  gsem.at[b]).wait()

        def start_scatter(b):
            pltpu.async_copy(rslot(b), acc.at[lbuf.at[b]], ssem.at[b],
                             add=True)
            pltpu.async_copy(ones, cacc.at[lbuf.at[b]], ssem.at[b], add=True)

        def wait_scatter(b):
            pltpu.make_async_copy(rslot(b), acc.at[lbuf.at[b]],
                                  ssem.at[b]).wait()
            pltpu.make_async_copy(ones, cacc.at[lbuf.at[b]],
                                  ssem.at[b]).wait()

        start_gather(0, 0)
        start_gather(1, 1)

        def ring_body(it, carry):
            for b in range(NBUF):
                kk = it * NBUF + b

                @pl.when(kk < NCHUNK)
                def _():
                    wait_gather(b)
                    start_scatter(b)

                bn = (b + 2) % NBUF

                @pl.when((kk >= 2) & (kk < NCHUNK + 2))
                def _():
                    wait_scatter(bn)

                @pl.when(kk + 2 < NCHUNK)
                def _():
                    start_gather(kk + 2, bn)
            return carry

        lax.fori_loop(0, (NCHUNK + NBUF - 1) // NBUF + 1, ring_body, 0)

        plsc.subcore_barrier()

        pltpu.sync_copy(acc.at[pl.ds(seg0, SEG_PER_TILE)],
                        sums_h.at[cid, pl.ds(seg0, SEG_PER_TILE)])
        pltpu.sync_copy(cacc.at[pl.ds(sid * CNT_PER_TILE, CNT_PER_TILE)],
                        cnt_h.at[cid, pl.ds(sid * CNT_PER_TILE, CNT_PER_TILE)])

    return k(node_input, labels)


def _combine_body(sums_ref, cnt_ref, out_ref):
    s = sums_ref[0, :NSEG] + sums_ref[1, :NSEG]
    c = jnp.maximum(cnt_ref[0, :NSEG] + cnt_ref[1, :NSEG], 1.0)
    out_ref[...] = s / c[:, None]


def kernel(node_input, labels):
    labels = labels.astype(jnp.int32)
    sums, cnts = _phase_a(node_input, labels)
    out = pl.pallas_call(
        _combine_body,
        out_shape=jax.ShapeDtypeStruct((NSEG, DIM), jnp.float32),
    )(sums, cnts)
    return out
